# Initial kernel scaffold; baseline (speedup 1.0000x reference)
import jax
import jax.numpy as jnp
from jax.experimental import pallas as pl


def kernel(x, down_src, down_dst, down_w, up_src, up_dst, up_w):
    raise NotImplementedError("write your pallas kernel here")


# speedup vs baseline: 28.3451x; 28.3451x over previous
import functools

import jax
import jax.numpy as jnp
from jax import lax
from jax.experimental import pallas as pl
from jax.experimental.pallas import tpu as pltpu
from jax.experimental.pallas import tpu_sc as plsc

N_FULL = 10000
N_COARSE = 2500
E = 320000
F = 128
B = 2
T = 2

NS = 16
C = 80
EPT = E // NS
NCH = EPT // C
NC_PAD = 2504
CSTRIDE = 152
OSTRIDE = 624


def _scale_rows(rows, wv):
    def body(g, carry):
        w16 = wv[pl.ds(g * 16, 16)]
        for lane in range(16):
            wb = w16.at[jnp.full((16,), lane, jnp.int32)].get(
                mode="promise_in_bounds")
            e = g * 16 + lane
            for fb in range(F // 16):
                sl = pl.ds(fb * 16, 16)
                rows[e, sl] = rows[e, sl] * wb
        return carry
    lax.fori_loop(0, C // 16, body, 0)


def _sc_body(x_hbm, dsrc, ddst, dw, usrc, udst, uw, out_hbm, coarse_hbm,
             sidx, didx, wv, rows, coarse_sh, full_sh, sem):
    b = lax.axis_index("c")
    t = lax.axis_index("s")
    xoff = (T * b + (T - 1)) * N_FULL

    z16 = jnp.zeros((16,), jnp.float32)

    def zrow(r, carry):
        for fb in range(F // 16):
            rows[r, pl.ds(fb * 16, 16)] = z16
        return carry
    lax.fori_loop(0, C, zrow, 0)

    pltpu.sync_copy(rows, coarse_sh.at[pl.ds(t * CSTRIDE, C)])
    pltpu.sync_copy(rows.at[pl.ds(0, CSTRIDE - C)],
                    coarse_sh.at[pl.ds(t * CSTRIDE + C, CSTRIDE - C)])

    @pl.when(t == 0)
    def _():
        pltpu.sync_copy(rows.at[pl.ds(0, NC_PAD - NS * CSTRIDE)],
                        coarse_sh.at[pl.ds(NS * CSTRIDE,
                                           NC_PAD - NS * CSTRIDE)])

    for k in range(8):
        pltpu.sync_copy(rows, full_sh.at[pl.ds(t * OSTRIDE + k * C, C)])
    plsc.subcore_barrier()

    ebase0 = t * EPT

    def down_chunk(k, carry):
        eb = ebase0 + k * C
        pltpu.sync_copy(dsrc.at[pl.ds(eb, C)], sidx)
        pltpu.sync_copy(ddst.at[pl.ds(eb, C)], didx)
        pltpu.sync_copy(dw.at[pl.ds(eb, C)], wv)
        for j in range(C // 16):
            sl = pl.ds(j * 16, 16)
            sidx[sl] = sidx[sl] + xoff
        pltpu.async_copy(x_hbm.at[sidx], rows, sem).wait()
        _scale_rows(rows, wv)
        pltpu.sync_copy(rows, coarse_sh.at[didx], add=True)
        return carry
    lax.fori_loop(0, NCH, down_chunk, 0)
    plsc.subcore_barrier()

    coff = b * NC_PAD
    pltpu.sync_copy(coarse_sh.at[pl.ds(t * CSTRIDE, CSTRIDE)],
                    coarse_hbm.at[pl.ds(coff + t * CSTRIDE, CSTRIDE)])

    @pl.when(t == 0)
    def _():
        tail = NC_PAD - NS * CSTRIDE
        pltpu.sync_copy(coarse_sh.at[pl.ds(NS * CSTRIDE, tail)],
                        coarse_hbm.at[pl.ds(coff + NS * CSTRIDE, tail)])
    plsc.subcore_barrier()

    def up_chunk(k, carry):
        eb = ebase0 + k * C
        pltpu.sync_copy(usrc.at[pl.ds(eb, C)], sidx)
        pltpu.sync_copy(udst.at[pl.ds(eb, C)], didx)
        pltpu.sync_copy(uw.at[pl.ds(eb, C)], wv)
        for j in range(C // 16):
            sl = pl.ds(j * 16, 16)
            sidx[sl] = sidx[sl] + coff
        pltpu.async_copy(coarse_hbm.at[sidx], rows, sem).wait()
        _scale_rows(rows, wv)
        pltpu.sync_copy(rows, full_sh.at[didx], add=True)
        return carry
    lax.fori_loop(0, NCH, up_chunk, 0)
    plsc.subcore_barrier()

    for k in range(8):
        src = pl.ds(t * OSTRIDE + k * C, C)
        pltpu.sync_copy(full_sh.at[src], rows)
        pltpu.sync_copy(rows,
                        out_hbm.at[pl.ds(b * N_FULL + t * OSTRIDE + k * C, C)])


_sc_call = functools.partial(
    pl.kernel,
    out_type=(jax.ShapeDtypeStruct((B * N_FULL, F), jnp.float32),
              jax.ShapeDtypeStruct((B * NC_PAD, F), jnp.float32)),
    mesh=plsc.VectorSubcoreMesh(core_axis_name="c", subcore_axis_name="s"),
    scratch_types=[
        pltpu.VMEM((C,), jnp.int32),
        pltpu.VMEM((C,), jnp.int32),
        pltpu.VMEM((C,), jnp.float32),
        pltpu.VMEM((C, F), jnp.float32),
        pltpu.VMEM_SHARED((NC_PAD, F), jnp.float32),
        pltpu.VMEM_SHARED((N_FULL, F), jnp.float32),
        pltpu.SemaphoreType.DMA,
    ],
)(_sc_body)


def kernel(x, down_src, down_dst, down_w, up_src, up_dst, up_w):
    bsz, tt, ens, n, f = x.shape
    x_flat = x.reshape(bsz * tt * ens * n, f)
    out, _ = _sc_call(x_flat, down_src, down_dst, down_w,
                      up_src, up_dst, up_w)
    return out.reshape(bsz, ens, n, f)


# speedup vs baseline: 49.0375x; 1.7300x over previous
import functools

import jax
import jax.numpy as jnp
from jax import lax
from jax.experimental import pallas as pl
from jax.experimental.pallas import tpu as pltpu
from jax.experimental.pallas import tpu_sc as plsc

N_FULL = 10000
N_COARSE = 2500
E = 320000
F = 128
B = 2
T = 2

NS = 16
C = 80
EPT = E // NS
EPB = 2000
NBLK = EPT // EPB
CPB = EPB // C
NC_PAD = 2504
CSTRIDE = 152
OSTRIDE = 624


def _scale_rows(rows, wbig, cbase):
    def body(g, carry):
        w16 = wbig[pl.ds(cbase + g * 16, 16)]
        for lane in range(16):
            wb = w16.at[jnp.full((16,), lane, jnp.int32)].get(
                mode="promise_in_bounds")
            e = g * 16 + lane
            for fb in range(F // 16):
                sl = pl.ds(fb * 16, 16)
                rows[e, sl] = rows[e, sl] * wb
        return carry
    lax.fori_loop(0, C // 16, body, 0)


def _sc_body(x_hbm, dsrc, ddst, dw, usrc, udst, uw, out_hbm,
             sbig, dbig, wbig, didx, rows, coarse_sh, full_sh, sem):
    b = lax.axis_index("c")
    t = lax.axis_index("s")
    xoff = (T * b + (T - 1)) * N_FULL

    z16 = jnp.zeros((16,), jnp.float32)

    def zrow(r, carry):
        for fb in range(F // 16):
            rows[r, pl.ds(fb * 16, 16)] = z16
        return carry
    lax.fori_loop(0, C, zrow, 0)

    pltpu.sync_copy(rows, coarse_sh.at[pl.ds(t * CSTRIDE, C)])
    pltpu.sync_copy(rows.at[pl.ds(0, CSTRIDE - C)],
                    coarse_sh.at[pl.ds(t * CSTRIDE + C, CSTRIDE - C)])

    @pl.when(t == 0)
    def _():
        tail = NC_PAD - NS * CSTRIDE
        pltpu.sync_copy(rows.at[pl.ds(0, tail)],
                        coarse_sh.at[pl.ds(NS * CSTRIDE, tail)])

    for k in range(8):
        pltpu.sync_copy(rows, full_sh.at[pl.ds(t * OSTRIDE + k * C, C)])
    plsc.subcore_barrier()

    ebase0 = t * EPT

    def _pass(src_hbm, dst_hbm, w_hbm, gather_from, acc_sh, idx_off):
        def block(blk, carry):
            eb = ebase0 + blk * EPB
            pltpu.sync_copy(src_hbm.at[pl.ds(eb, EPB)], sbig)
            pltpu.sync_copy(dst_hbm.at[pl.ds(eb, EPB)], dbig)
            pltpu.sync_copy(w_hbm.at[pl.ds(eb, EPB)], wbig)

            def off(j, carry2):
                sl = pl.ds(j * 16, 16)
                sbig[sl] = sbig[sl] + idx_off
                return carry2
            lax.fori_loop(0, EPB // 16, off, 0)

            def chunk(ck, carry2):
                cb = ck * C
                pltpu.async_copy(gather_from.at[sbig.at[pl.ds(cb, C)]],
                                 rows, sem).wait()
                _scale_rows(rows, wbig, cb)
                for j in range(C // 16):
                    sl = pl.ds(j * 16, 16)
                    didx[sl] = dbig[pl.ds(cb + j * 16, 16)]
                pltpu.sync_copy(rows, acc_sh.at[didx], add=True)
                return carry2
            lax.fori_loop(0, CPB, chunk, 0)
            return carry
        lax.fori_loop(0, NBLK, block, 0)

    _pass(dsrc, ddst, dw, x_hbm, coarse_sh, xoff)
    plsc.subcore_barrier()

    _pass(usrc, udst, uw, coarse_sh, full_sh, 0)
    plsc.subcore_barrier()

    for k in range(8):
        src = pl.ds(t * OSTRIDE + k * C, C)
        pltpu.sync_copy(full_sh.at[src], rows)
        pltpu.sync_copy(rows,
                        out_hbm.at[pl.ds(b * N_FULL + t * OSTRIDE + k * C, C)])


_sc_call = functools.partial(
    pl.kernel,
    out_type=jax.ShapeDtypeStruct((B * N_FULL, F), jnp.float32),
    mesh=plsc.VectorSubcoreMesh(core_axis_name="c", subcore_axis_name="s"),
    scratch_types=[
        pltpu.VMEM((EPB,), jnp.int32),
        pltpu.VMEM((EPB,), jnp.int32),
        pltpu.VMEM((EPB,), jnp.float32),
        pltpu.VMEM((C,), jnp.int32),
        pltpu.VMEM((C, F), jnp.float32),
        pltpu.VMEM_SHARED((NC_PAD, F), jnp.float32),
        pltpu.VMEM_SHARED((N_FULL, F), jnp.float32),
        pltpu.SemaphoreType.DMA,
    ],
)(_sc_body)


def kernel(x, down_src, down_dst, down_w, up_src, up_dst, up_w):
    bsz, tt, ens, n, f = x.shape
    x_flat = x.reshape(bsz * tt * ens * n, f)
    out = _sc_call(x_flat, down_src, down_dst, down_w, up_src, up_dst, up_w)
    return out.reshape(bsz, ens, n, f)


# speedup vs baseline: 66.2105x; 1.3502x over previous
import functools

import jax
import jax.numpy as jnp
from jax import lax
from jax.experimental import pallas as pl
from jax.experimental.pallas import tpu as pltpu
from jax.experimental.pallas import tpu_sc as plsc

N_FULL = 10000
N_COARSE = 2500
E = 320000
F = 128
B = 2
T = 2

NS = 16
C = 80
EPT = E // NS
EPB = 800
NBLK = EPT // EPB
CPB = EPB // C
NC_PAD = 2504
CSTRIDE = 152
OSTRIDE = 624


def _scale_rows(rows, wbig, cbase):
    def body(g, carry):
        w16 = wbig[pl.ds(cbase + g * 16, 16)]
        for lane in range(16):
            wb = w16.at[jnp.full((16,), lane, jnp.int32)].get(
                mode="promise_in_bounds")
            e = g * 16 + lane
            for fb in range(F // 16):
                sl = pl.ds(fb * 16, 16)
                rows[e, sl] = rows[e, sl] * wb
        return carry
    lax.fori_loop(0, C // 16, body, 0)


def _sc_body(x_hbm, dsrc, ddst, dw, usrc, udst, uw, out_hbm,
             sbig, dbig, wbig, didx, rows, rows1, coarse_sh, full_sh,
             sem, sem1):
    b = lax.axis_index("c")
    t = lax.axis_index("s")
    xoff = (T * b + (T - 1)) * N_FULL

    z16 = jnp.zeros((16,), jnp.float32)

    def zrow(r, carry):
        for fb in range(F // 16):
            rows[r, pl.ds(fb * 16, 16)] = z16
        return carry
    lax.fori_loop(0, C, zrow, 0)

    pltpu.sync_copy(rows, coarse_sh.at[pl.ds(t * CSTRIDE, C)])
    pltpu.sync_copy(rows.at[pl.ds(0, CSTRIDE - C)],
                    coarse_sh.at[pl.ds(t * CSTRIDE + C, CSTRIDE - C)])

    @pl.when(t == 0)
    def _():
        tail = NC_PAD - NS * CSTRIDE
        pltpu.sync_copy(rows.at[pl.ds(0, tail)],
                        coarse_sh.at[pl.ds(NS * CSTRIDE, tail)])

    for k in range(8):
        pltpu.sync_copy(rows, full_sh.at[pl.ds(t * OSTRIDE + k * C, C)])
    plsc.subcore_barrier()

    ebase0 = t * EPT

    def _pass(src_hbm, dst_hbm, w_hbm, gather_from, acc_sh, idx_off):
        def block(blk, carry):
            eb = ebase0 + blk * EPB
            pltpu.sync_copy(src_hbm.at[pl.ds(eb, EPB)], sbig)
            pltpu.sync_copy(dst_hbm.at[pl.ds(eb, EPB)], dbig)
            pltpu.sync_copy(w_hbm.at[pl.ds(eb, EPB)], wbig)

            def off(j, carry2):
                sl = pl.ds(j * 16, 16)
                sbig[sl] = sbig[sl] + idx_off
                return carry2
            lax.fori_loop(0, EPB // 16, off, 0)

            def start_gather(ck, buf, sm):
                pltpu.async_copy(
                    gather_from.at[sbig.at[pl.ds(ck * C, C)]], buf, sm)

            def process(ck, buf, sm):
                pltpu.make_async_copy(
                    gather_from.at[sbig.at[pl.ds(ck * C, C)]], buf, sm
                ).wait()
                cb = ck * C
                _scale_rows(buf, wbig, cb)
                for j in range(C // 16):
                    sl = pl.ds(j * 16, 16)
                    didx[sl] = dbig[pl.ds(cb + j * 16, 16)]
                pltpu.sync_copy(buf, acc_sh.at[didx], add=True)

            start_gather(0, rows, sem)

            def pair(p, carry2):
                c0 = p * 2
                start_gather(c0 + 1, rows1, sem1)
                process(c0, rows, sem)

                @pl.when(c0 + 2 < CPB)
                def _():
                    start_gather(c0 + 2, rows, sem)
                process(c0 + 1, rows1, sem1)
                return carry2
            lax.fori_loop(0, CPB // 2, pair, 0)
            return carry
        lax.fori_loop(0, NBLK, block, 0)

    _pass(dsrc, ddst, dw, x_hbm, coarse_sh, xoff)
    plsc.subcore_barrier()

    _pass(usrc, udst, uw, coarse_sh, full_sh, 0)
    plsc.subcore_barrier()

    for k in range(8):
        src = pl.ds(t * OSTRIDE + k * C, C)
        pltpu.sync_copy(full_sh.at[src], rows)
        pltpu.sync_copy(rows,
                        out_hbm.at[pl.ds(b * N_FULL + t * OSTRIDE + k * C, C)])


_sc_call = functools.partial(
    pl.kernel,
    out_type=jax.ShapeDtypeStruct((B * N_FULL, F), jnp.float32),
    mesh=plsc.VectorSubcoreMesh(core_axis_name="c", subcore_axis_name="s"),
    scratch_types=[
        pltpu.VMEM((EPB,), jnp.int32),
        pltpu.VMEM((EPB,), jnp.int32),
        pltpu.VMEM((EPB,), jnp.float32),
        pltpu.VMEM((C,), jnp.int32),
        pltpu.VMEM((C, F), jnp.float32),
        pltpu.VMEM((C, F), jnp.float32),
        pltpu.VMEM_SHARED((NC_PAD, F), jnp.float32),
        pltpu.VMEM_SHARED((N_FULL, F), jnp.float32),
        pltpu.SemaphoreType.DMA,
        pltpu.SemaphoreType.DMA,
    ],
)(_sc_body)


def kernel(x, down_src, down_dst, down_w, up_src, up_dst, up_w):
    bsz, tt, ens, n, f = x.shape
    x_flat = x.reshape(bsz * tt * ens * n, f)
    out = _sc_call(x_flat, down_src, down_dst, down_w, up_src, up_dst, up_w)
    return out.reshape(bsz, ens, n, f)
